# baseline (speedup 1.0000x reference)
import functools

import jax
import jax.numpy as jnp
from jax import lax
from jax.experimental import pallas as pl
from jax.experimental.pallas import tpu as pltpu
from jax.experimental.pallas import tpu_sc as plsc

N = 10000
NPAD = 10240
R = 3
E = 320000
E_PAD = 327680
CH = 128
K = 4
GRP = K * CH
NC = 2
NS = 16
ROWS = R * NPAD
RPT = ROWS // NS
BN = 1024
H1 = 50
H2 = 20


def _seg_pass(tab, src, dst, et, mult, qoff):
    epw = E_PAD // NS
    G = epw // GRP
    mesh = plsc.VectorSubcoreMesh(core_axis_name="c", subcore_axis_name="s")

    scratch = [
        pltpu.VMEM((2, GRP), jnp.int32),
        pltpu.VMEM((2, GRP), jnp.int32),
        pltpu.VMEM((2, GRP), jnp.int32),
        pltpu.VMEM((2, K, CH), jnp.int32),
        pltpu.VMEM((2, K, CH), jnp.int32),
        pltpu.VMEM((2, K, CH, 32), jnp.float32),
        pltpu.VMEM((CH, 32), jnp.float32),
        pltpu.VMEM_SHARED((ROWS, 32), jnp.float32),
        pltpu.SemaphoreType.DMA,
        pltpu.SemaphoreType.DMA,
        pltpu.SemaphoreType.DMA,
    ]
    @functools.partial(
        pl.kernel,
        out_type=jax.ShapeDtypeStruct((NC, ROWS, 32), jnp.float32),
        mesh=mesh,
        compiler_params=pltpu.CompilerParams(use_tc_tiling_on_sc=False),
        scratch_types=scratch,
    )
    def k(*refs):
        (tab_r, src_r, dst_r, et_r, out_s, svec, dvec, evec, gidx,
         sidx, rows, zbuf, accum, sem_i, sem_g, sem_s) = refs
        cid = lax.axis_index("c")
        sid = lax.axis_index("s")
        ebase = sid * epw
        qof = qoff + cid

        def zrow(i, _):
            zbuf[i, pl.ds(0, 16)] = jnp.zeros((16,), jnp.float32)
            zbuf[i, pl.ds(16, 16)] = jnp.zeros((16,), jnp.float32)
            return _
        lax.fori_loop(0, CH, zrow, None)

        def zfire(t, _):
            pltpu.async_copy(zbuf, accum.at[pl.ds(sid * RPT + t * CH, CH)],
                             sem_i)
            return _
        lax.fori_loop(0, RPT // CH, zfire, None)

        def zwait(t, _):
            pltpu.make_async_copy(zbuf, accum.at[pl.ds(sid * RPT, CH)],
                                  sem_i).wait()
            return _
        lax.fori_loop(0, RPT // CH, zwait, None)
        plsc.subcore_barrier()

        def idx_fire(g, pb):
            b = ebase + g * GRP
            pltpu.async_copy(src_r.at[pl.ds(b, GRP)], svec.at[pb], sem_i)
            pltpu.async_copy(dst_r.at[pl.ds(b, GRP)], dvec.at[pb], sem_i)
            pltpu.async_copy(et_r.at[pl.ds(b, GRP)], evec.at[pb], sem_i)

        def idx_wait(pb):
            pltpu.make_async_copy(src_r.at[pl.ds(0, GRP)], svec.at[pb],
                                  sem_i).wait()
            pltpu.make_async_copy(dst_r.at[pl.ds(0, GRP)], dvec.at[pb],
                                  sem_i).wait()
            pltpu.make_async_copy(et_r.at[pl.ds(0, GRP)], evec.at[pb],
                                  sem_i).wait()

        def idx_compute(pb):
            for j in range(K):
                for i in range(CH // 16):
                    o = j * CH + i * 16
                    sl = pl.ds(i * 16, 16)
                    sv = svec[pb, pl.ds(o, 16)]
                    gidx[pb, j, sl] = sv * mult + qof
                    d = dvec[pb, pl.ds(o, 16)]
                    e = evec[pb, pl.ds(o, 16)]
                    sidx[pb, j, sl] = e * NPAD + d

        def gather_fire(pb, j):
            pltpu.async_copy(tab_r.at[gidx.at[pb, j]], rows.at[pb, j],
                             sem_g)

        def gather_wait(pb, j):
            pltpu.make_async_copy(tab_r.at[gidx.at[pb, j]], rows.at[pb, j],
                                  sem_g).wait()

        def scat_fire(pb, j):
            pltpu.async_copy(rows.at[pb, j], accum.at[sidx.at[pb, j]],
                             sem_s, add=True)

        def scat_wait(pb, j):
            pltpu.make_async_copy(rows.at[pb, j], accum.at[sidx.at[pb, j]],
                                  sem_s).wait()

        idx_fire(0, 0)
        idx_wait(0)
        idx_compute(0)
        idx_fire(1, 1)
        for j in range(K):
            gather_fire(0, j)
        idx_wait(1)
        idx_compute(1)
        for j in range(K):
            gather_wait(0, j)
            scat_fire(0, j)
        idx_fire(2, 0)
        for j in range(K):
            gather_fire(1, j)

        def pair(t, _):
            for pb in range(2):
                g = 2 + 2 * t + pb
                idx_wait(pb)
                for j in range(K):
                    scat_wait(pb, j)
                idx_compute(pb)
                for j in range(K):
                    gather_wait(1 - pb, j)
                    scat_fire(1 - pb, j)
                gnext = jnp.minimum(g + 1, G - 1)
                idx_fire(gnext, 1 - pb)
                for j in range(K):
                    gather_fire(pb, j)
            return _
        lax.fori_loop(0, (G - 2) // 2, pair, None)

        idx_wait(0)
        for j in range(K):
            scat_wait(0, j)
        for j in range(K):
            gather_wait(1, j)
            scat_fire(1, j)
        for j in range(K):
            scat_wait(1, j)
        plsc.subcore_barrier()

        sl = pl.ds(sid * RPT, RPT)
        pltpu.sync_copy(accum.at[sl], out_s.at[cid, sl])

    return k(tab, src, dst, et)


def _cnt_pass(dst, et):
    epw = E_PAD // (NC * NS)
    G = epw // GRP
    mesh = plsc.VectorSubcoreMesh(core_axis_name="c", subcore_axis_name="s")

    @functools.partial(
        pl.kernel,
        out_type=jax.ShapeDtypeStruct((NC, ROWS, 16), jnp.float32),
        mesh=mesh,
        compiler_params=pltpu.CompilerParams(use_tc_tiling_on_sc=False),
        scratch_types=[
            pltpu.VMEM((2, GRP), jnp.int32),
            pltpu.VMEM((2, GRP), jnp.int32),
            pltpu.VMEM((2, K, CH), jnp.int32),
            pltpu.VMEM((CH, 16), jnp.float32),
            pltpu.VMEM((CH, 16), jnp.float32),
            pltpu.VMEM_SHARED((ROWS, 16), jnp.float32),
            pltpu.SemaphoreType.DMA,
            pltpu.SemaphoreType.DMA,
        ],
    )
    def k(dst_r, et_r, out_c, dvec, evec, sidx, ones, zbuf, accum, sem_i,
          sem_c):
        cid = lax.axis_index("c")
        sid = lax.axis_index("s")
        ebase = (sid * NC + cid) * epw

        def zrow(i, _):
            ones[i, pl.ds(0, 16)] = jnp.ones((16,), jnp.float32)
            zbuf[i, pl.ds(0, 16)] = jnp.zeros((16,), jnp.float32)
            return _
        lax.fori_loop(0, CH, zrow, None)

        def zfire(t, _):
            pltpu.async_copy(zbuf, accum.at[pl.ds(sid * RPT + t * CH, CH)],
                             sem_i)
            return _
        lax.fori_loop(0, RPT // CH, zfire, None)

        def zwait(t, _):
            pltpu.make_async_copy(zbuf, accum.at[pl.ds(sid * RPT, CH)],
                                  sem_i).wait()
            return _
        lax.fori_loop(0, RPT // CH, zwait, None)
        plsc.subcore_barrier()

        def idx_fire(g, pb):
            b = ebase + g * GRP
            pltpu.async_copy(dst_r.at[pl.ds(b, GRP)], dvec.at[pb], sem_i)
            pltpu.async_copy(et_r.at[pl.ds(b, GRP)], evec.at[pb], sem_i)

        def idx_wait(pb):
            pltpu.make_async_copy(dst_r.at[pl.ds(0, GRP)], dvec.at[pb],
                                  sem_i).wait()
            pltpu.make_async_copy(et_r.at[pl.ds(0, GRP)], evec.at[pb],
                                  sem_i).wait()

        def sidx_compute(pb):
            for j in range(K):
                for i in range(CH // 16):
                    o = j * CH + i * 16
                    d = dvec[pb, pl.ds(o, 16)]
                    e = evec[pb, pl.ds(o, 16)]
                    sidx[pb, j, pl.ds(i * 16, 16)] = e * NPAD + d

        def scat_fire(pb, j):
            pltpu.async_copy(ones, accum.at[sidx.at[pb, j]], sem_c,
                             add=True)

        def scat_wait(pb, j):
            pltpu.make_async_copy(ones, accum.at[sidx.at[pb, j]],
                                  sem_c).wait()

        idx_fire(0, 0)
        idx_wait(0)
        sidx_compute(0)
        idx_fire(1, 1)
        for j in range(K):
            scat_fire(0, j)
        idx_wait(1)
        sidx_compute(1)
        for j in range(K):
            scat_fire(1, j)
        idx_fire(2, 0)

        def pair(t, _):
            for pb in range(2):
                g = 2 + 2 * t + pb
                idx_wait(pb)
                for j in range(K):
                    scat_wait(pb, j)
                sidx_compute(pb)
                for j in range(K):
                    scat_fire(pb, j)
                gnext = jnp.minimum(g + 1, G - 1)
                idx_fire(gnext, 1 - pb)
            return _
        lax.fori_loop(0, (G - 2) // 2, pair, None)

        idx_wait(0)
        for j in range(K):
            scat_wait(0, j)
        for j in range(K):
            scat_wait(1, j)
        plsc.subcore_barrier()

        sl = pl.ds(sid * RPT, RPT)
        pltpu.sync_copy(accum.at[sl], out_c.at[cid, sl])

    return k(dst, et)


def _gates_kernel(x, sums_list, cnts, basis, comp, root, bx, bh, dout,
                  lin_w=None, lin_b=None):
    din = x.shape[1]
    grid = (NPAD // BN,)
    final = lin_w is not None
    out_w = 1 if final else dout
    nsum = len(sums_list)

    def body(*refs):
        i = 1
        x_ref = refs[0]
        sums_refs = refs[i:i + 3 * nsum]
        i += 3 * nsum
        cnt_refs = refs[i:i + 3]
        i += 3
        basis_ref, comp_ref, root_ref, bx_ref, bh_ref = refs[i:i + 5]
        i += 5
        if final:
            lw_ref, lb_ref = refs[i:i + 2]
        out_ref = refs[-1]

        xb = x_ref[...]
        pieces = [xb]
        for r in range(R):
            c = cnt_refs[r][0, :, 0] + cnt_refs[r][1, :, 0]
            iv = (1.0 / jnp.maximum(c, 1.0))[:, None]
            for a in range(nsum):
                sref = sums_refs[a * 3 + r]
                pieces.append(sref[0] * iv)
                pieces.append(sref[1] * iv)
        mcat = jnp.concatenate(pieces, axis=1)

        wcols = []
        bparts = []
        for k in (0, 2, 3):
            rows = [root_ref[k]]
            for r in range(R):
                rows.append(comp_ref[k, r, 0] * basis_ref[k, 0]
                            + comp_ref[k, r, 1] * basis_ref[k, 1]
                            + comp_ref[k, r, 2] * basis_ref[k, 2])
            wcols.append(jnp.concatenate(rows, axis=0))
            bparts.append(bx_ref[k] + bh_ref[k])
        wall = jnp.concatenate(wcols, axis=1)
        bias = jnp.concatenate(bparts, axis=0)

        z = jnp.dot(mcat, wall, preferred_element_type=jnp.float32) + bias
        gi = jax.nn.sigmoid(z[:, :dout])
        gt = jnp.tanh(z[:, dout:2 * dout])
        go = jax.nn.sigmoid(z[:, 2 * dout:])
        h = go * jnp.tanh(gi * gt)
        if final:
            h = jnp.maximum(h, 0.0)
            out_ref[...] = (jnp.dot(h, lw_ref[...],
                                    preferred_element_type=jnp.float32)
                            + lb_ref[0])
        else:
            out_ref[...] = h

    full = lambda s: pl.BlockSpec(s, lambda i: (0,) * len(s))
    in_specs = [pl.BlockSpec((BN, din), lambda i: (i, 0))]
    operands = [x]
    for arr in sums_list:
        for r in range(R):
            in_specs.append(pl.BlockSpec(
                (NC, BN, 32), lambda i, r=r: (0, r * (NPAD // BN) + i, 0)))
            operands.append(arr)
    for r in range(R):
        in_specs.append(pl.BlockSpec(
            (NC, BN, 16), lambda i, r=r: (0, r * (NPAD // BN) + i, 0)))
        operands.append(cnts)
    in_specs += [full(basis.shape), full(comp.shape), full(root.shape),
                 full(bx.shape), full(bh.shape)]
    operands += [basis, comp, root, bx, bh]
    if final:
        in_specs += [full(lin_w.shape), full(lin_b.shape)]
        operands += [lin_w, lin_b]

    return pl.pallas_call(
        body,
        grid=grid,
        in_specs=in_specs,
        out_specs=pl.BlockSpec((BN, out_w), lambda i: (i, 0)),
        out_shape=jax.ShapeDtypeStruct((N, out_w), jnp.float32),
    )(*operands)


def kernel(x, edge_index, edge_weight, l1x_basis, l1x_comp, l1x_root,
           l1x_bias, l1h_basis, l1h_comp, l1h_root, l1h_bias, l2x_basis,
           l2x_comp, l2x_root, l2x_bias, l2h_basis, l2h_comp, l2h_root,
           l2h_bias, lin_w, lin_b):
    npad_e = E_PAD - E
    src = jnp.concatenate([edge_index[0], jnp.zeros((npad_e,), jnp.int32)])
    dst = jnp.concatenate([edge_index[1],
                           jnp.full((npad_e,), N, jnp.int32)])
    et = jnp.concatenate([edge_weight, jnp.zeros((npad_e,), jnp.int32)])

    b1 = jnp.pad(l1x_basis, ((0, 0), (0, 0), (0, 0), (0, 14)))
    r1 = jnp.pad(l1x_root, ((0, 0), (0, 0), (0, 14)))
    bx1 = jnp.pad(l1x_bias, ((0, 0), (0, 14)))
    bh1 = jnp.pad(l1h_bias, ((0, 0), (0, 14)))
    b2 = jnp.pad(l2x_basis, ((0, 0), (0, 0), (0, 14), (0, 12)))
    r2 = jnp.pad(l2x_root, ((0, 0), (0, 14), (0, 12)))
    bx2 = jnp.pad(l2x_bias, ((0, 0), (0, 12)))
    bh2 = jnp.pad(l2h_bias, ((0, 0), (0, 12)))
    lw = jnp.pad(lin_w, ((0, 12), (0, 0)))

    cnts = _cnt_pass(dst, et)
    xs4 = x.reshape(4 * N, 32)
    s1a = _seg_pass(xs4, src, dst, et, 4, 0)
    s1b = _seg_pass(xs4, src, dst, et, 4, 2)
    h1p = _gates_kernel(x, [s1a, s1b], cnts, b1, l1x_comp, r1, bx1, bh1,
                        dout=64)

    sums2 = _seg_pass(h1p.reshape(2 * N, 32), src, dst, et, 2, 0)
    out = _gates_kernel(h1p, [sums2], cnts, b2, l2x_comp, r2, bx2, bh2,
                        dout=32, lin_w=lw, lin_b=lin_b)
    return out


# speedup vs baseline: 1.1174x; 1.1174x over previous
import functools

import jax
import jax.numpy as jnp
from jax import lax
from jax.experimental import pallas as pl
from jax.experimental.pallas import tpu as pltpu
from jax.experimental.pallas import tpu_sc as plsc

N = 10000
NPAD = 10240
R = 3
E = 320000
E_PAD = 327680
CH = 128
K = 4
GRP = K * CH
NC = 2
NS = 16
ROWS = R * NPAD
RPT = ROWS // NS
BN = 1024
H1 = 50
H2 = 20


def _seg_pass(tab_lo, tab_hi, src, dst, et):
    epw = E_PAD // NS
    G = epw // GRP
    mesh = plsc.VectorSubcoreMesh(core_axis_name="c", subcore_axis_name="s")

    scratch = [
        pltpu.VMEM((2, GRP), jnp.int32),
        pltpu.VMEM((2, GRP), jnp.int32),
        pltpu.VMEM((2, GRP), jnp.int32),
        pltpu.VMEM((2, K, CH), jnp.int32),
        pltpu.VMEM((2, K, CH, 32), jnp.float32),
        pltpu.VMEM((CH, 32), jnp.float32),
        pltpu.VMEM_SHARED((ROWS, 32), jnp.float32),
        pltpu.SemaphoreType.DMA,
        pltpu.SemaphoreType.DMA,
        pltpu.SemaphoreType.DMA,
    ]
    @functools.partial(
        pl.kernel,
        out_type=jax.ShapeDtypeStruct((NC, ROWS, 32), jnp.float32),
        mesh=mesh,
        compiler_params=pltpu.CompilerParams(use_tc_tiling_on_sc=False),
        scratch_types=scratch,
    )
    def k(*refs):
        (tlo, thi, src_r, dst_r, et_r, out_s, gidx, dvec, evec,
         sidx, rows, zbuf, accum, sem_i, sem_g, sem_s) = refs
        cid = lax.axis_index("c")
        sid = lax.axis_index("s")
        ebase = sid * epw

        def zrow(i, _):
            zbuf[i, pl.ds(0, 16)] = jnp.zeros((16,), jnp.float32)
            zbuf[i, pl.ds(16, 16)] = jnp.zeros((16,), jnp.float32)
            return _
        lax.fori_loop(0, CH, zrow, None)

        def zfire(t, _):
            pltpu.async_copy(zbuf, accum.at[pl.ds(sid * RPT + t * CH, CH)],
                             sem_i)
            return _
        lax.fori_loop(0, RPT // CH, zfire, None)

        def zwait(t, _):
            pltpu.make_async_copy(zbuf, accum.at[pl.ds(sid * RPT, CH)],
                                  sem_i).wait()
            return _
        lax.fori_loop(0, RPT // CH, zwait, None)
        plsc.subcore_barrier()

        def idx_fire(g, pb):
            b = ebase + g * GRP
            pltpu.async_copy(src_r.at[pl.ds(b, GRP)], gidx.at[pb], sem_i)
            pltpu.async_copy(dst_r.at[pl.ds(b, GRP)], dvec.at[pb], sem_i)
            pltpu.async_copy(et_r.at[pl.ds(b, GRP)], evec.at[pb], sem_i)

        def idx_wait(pb):
            pltpu.make_async_copy(src_r.at[pl.ds(0, GRP)], gidx.at[pb],
                                  sem_i).wait()
            pltpu.make_async_copy(dst_r.at[pl.ds(0, GRP)], dvec.at[pb],
                                  sem_i).wait()
            pltpu.make_async_copy(et_r.at[pl.ds(0, GRP)], evec.at[pb],
                                  sem_i).wait()

        def idx_compute(pb):
            for j in range(K):
                for i in range(CH // 16):
                    o = j * CH + i * 16
                    d = dvec[pb, pl.ds(o, 16)]
                    e = evec[pb, pl.ds(o, 16)]
                    sidx[pb, j, pl.ds(i * 16, 16)] = e * NPAD + d

        def gather_fire(pb, j):
            idx = gidx.at[pb, pl.ds(j * CH, CH)]

            @pl.when(cid == 0)
            def _():
                pltpu.async_copy(tlo.at[idx], rows.at[pb, j], sem_g)

            @pl.when(cid == 1)
            def _():
                pltpu.async_copy(thi.at[idx], rows.at[pb, j], sem_g)

        def gather_wait(pb, j):
            idx = gidx.at[pb, pl.ds(j * CH, CH)]
            pltpu.make_async_copy(tlo.at[idx], rows.at[pb, j], sem_g).wait()

        def scat_fire(pb, j):
            pltpu.async_copy(rows.at[pb, j], accum.at[sidx.at[pb, j]],
                             sem_s, add=True)

        def scat_wait(pb, j):
            pltpu.make_async_copy(rows.at[pb, j], accum.at[sidx.at[pb, j]],
                                  sem_s).wait()

        idx_fire(0, 0)
        idx_wait(0)
        idx_compute(0)
        idx_fire(1, 1)
        for j in range(K):
            gather_fire(0, j)
        idx_wait(1)
        idx_compute(1)
        for j in range(K):
            gather_wait(0, j)
            scat_fire(0, j)
        idx_fire(2, 0)
        for j in range(K):
            gather_fire(1, j)

        def pair(t, _):
            for pb in range(2):
                g = 2 + 2 * t + pb
                idx_wait(pb)
                for j in range(K):
                    scat_wait(pb, j)
                idx_compute(pb)
                for j in range(K):
                    gather_wait(1 - pb, j)
                    scat_fire(1 - pb, j)
                gnext = jnp.minimum(g + 1, G - 1)
                idx_fire(gnext, 1 - pb)
                for j in range(K):
                    gather_fire(pb, j)
            return _
        lax.fori_loop(0, (G - 2) // 2, pair, None)

        idx_wait(0)
        for j in range(K):
            scat_wait(0, j)
        for j in range(K):
            gather_wait(1, j)
            scat_fire(1, j)
        for j in range(K):
            scat_wait(1, j)
        plsc.subcore_barrier()

        sl = pl.ds(sid * RPT, RPT)
        pltpu.sync_copy(accum.at[sl], out_s.at[cid, sl])

    return k(tab_lo, tab_hi, src, dst, et)


def _cnt_pass(dst, et):
    epw = E_PAD // (NC * NS)
    G = epw // GRP
    mesh = plsc.VectorSubcoreMesh(core_axis_name="c", subcore_axis_name="s")

    @functools.partial(
        pl.kernel,
        out_type=jax.ShapeDtypeStruct((NC, ROWS, 16), jnp.float32),
        mesh=mesh,
        compiler_params=pltpu.CompilerParams(use_tc_tiling_on_sc=False),
        scratch_types=[
            pltpu.VMEM((2, GRP), jnp.int32),
            pltpu.VMEM((2, GRP), jnp.int32),
            pltpu.VMEM((2, K, CH), jnp.int32),
            pltpu.VMEM((CH, 16), jnp.float32),
            pltpu.VMEM((CH, 16), jnp.float32),
            pltpu.VMEM_SHARED((ROWS, 16), jnp.float32),
            pltpu.SemaphoreType.DMA,
            pltpu.SemaphoreType.DMA,
        ],
    )
    def k(dst_r, et_r, out_c, dvec, evec, sidx, ones, zbuf, accum, sem_i,
          sem_c):
        cid = lax.axis_index("c")
        sid = lax.axis_index("s")
        ebase = (sid * NC + cid) * epw

        def zrow(i, _):
            ones[i, pl.ds(0, 16)] = jnp.ones((16,), jnp.float32)
            zbuf[i, pl.ds(0, 16)] = jnp.zeros((16,), jnp.float32)
            return _
        lax.fori_loop(0, CH, zrow, None)

        def zfire(t, _):
            pltpu.async_copy(zbuf, accum.at[pl.ds(sid * RPT + t * CH, CH)],
                             sem_i)
            return _
        lax.fori_loop(0, RPT // CH, zfire, None)

        def zwait(t, _):
            pltpu.make_async_copy(zbuf, accum.at[pl.ds(sid * RPT, CH)],
                                  sem_i).wait()
            return _
        lax.fori_loop(0, RPT // CH, zwait, None)
        plsc.subcore_barrier()

        def idx_fire(g, pb):
            b = ebase + g * GRP
            pltpu.async_copy(dst_r.at[pl.ds(b, GRP)], dvec.at[pb], sem_i)
            pltpu.async_copy(et_r.at[pl.ds(b, GRP)], evec.at[pb], sem_i)

        def idx_wait(pb):
            pltpu.make_async_copy(dst_r.at[pl.ds(0, GRP)], dvec.at[pb],
                                  sem_i).wait()
            pltpu.make_async_copy(et_r.at[pl.ds(0, GRP)], evec.at[pb],
                                  sem_i).wait()

        def sidx_compute(pb):
            for j in range(K):
                for i in range(CH // 16):
                    o = j * CH + i * 16
                    d = dvec[pb, pl.ds(o, 16)]
                    e = evec[pb, pl.ds(o, 16)]
                    sidx[pb, j, pl.ds(i * 16, 16)] = e * NPAD + d

        def scat_fire(pb, j):
            pltpu.async_copy(ones, accum.at[sidx.at[pb, j]], sem_c,
                             add=True)

        def scat_wait(pb, j):
            pltpu.make_async_copy(ones, accum.at[sidx.at[pb, j]],
                                  sem_c).wait()

        idx_fire(0, 0)
        idx_wait(0)
        sidx_compute(0)
        idx_fire(1, 1)
        for j in range(K):
            scat_fire(0, j)
        idx_wait(1)
        sidx_compute(1)
        for j in range(K):
            scat_fire(1, j)
        idx_fire(2, 0)

        def pair(t, _):
            for pb in range(2):
                g = 2 + 2 * t + pb
                idx_wait(pb)
                for j in range(K):
                    scat_wait(pb, j)
                sidx_compute(pb)
                for j in range(K):
                    scat_fire(pb, j)
                gnext = jnp.minimum(g + 1, G - 1)
                idx_fire(gnext, 1 - pb)
            return _
        lax.fori_loop(0, (G - 2) // 2, pair, None)

        idx_wait(0)
        for j in range(K):
            scat_wait(0, j)
        for j in range(K):
            scat_wait(1, j)
        plsc.subcore_barrier()

        sl = pl.ds(sid * RPT, RPT)
        pltpu.sync_copy(accum.at[sl], out_c.at[cid, sl])

    return k(dst, et)


def _gates_kernel(x, sums_list, cnts, basis, comp, root, bx, bh, dout,
                  lin_w=None, lin_b=None):
    din = x.shape[1]
    grid = (NPAD // BN,)
    final = lin_w is not None
    out_w = 1 if final else dout
    nsum = len(sums_list)

    def body(*refs):
        i = 1
        x_ref = refs[0]
        sums_refs = refs[i:i + 3 * nsum]
        i += 3 * nsum
        cnt_refs = refs[i:i + 3]
        i += 3
        basis_ref, comp_ref, root_ref, bx_ref, bh_ref = refs[i:i + 5]
        i += 5
        if final:
            lw_ref, lb_ref = refs[i:i + 2]
        out_ref = refs[-1]

        xb = x_ref[...]
        pieces = [xb]
        for r in range(R):
            c = cnt_refs[r][0, :, 0] + cnt_refs[r][1, :, 0]
            iv = (1.0 / jnp.maximum(c, 1.0))[:, None]
            for a in range(nsum):
                sref = sums_refs[a * 3 + r]
                pieces.append(sref[0] * iv)
                pieces.append(sref[1] * iv)
        mcat = jnp.concatenate(pieces, axis=1)

        wcols = []
        bparts = []
        for k in (0, 2, 3):
            rows = [root_ref[k]]
            for r in range(R):
                rows.append(comp_ref[k, r, 0] * basis_ref[k, 0]
                            + comp_ref[k, r, 1] * basis_ref[k, 1]
                            + comp_ref[k, r, 2] * basis_ref[k, 2])
            wcols.append(jnp.concatenate(rows, axis=0))
            bparts.append(bx_ref[k] + bh_ref[k])
        wall = jnp.concatenate(wcols, axis=1)
        bias = jnp.concatenate(bparts, axis=0)

        z = jnp.dot(mcat, wall, preferred_element_type=jnp.float32) + bias
        gi = jax.nn.sigmoid(z[:, :dout])
        gt = jnp.tanh(z[:, dout:2 * dout])
        go = jax.nn.sigmoid(z[:, 2 * dout:])
        h = go * jnp.tanh(gi * gt)
        if final:
            h = jnp.maximum(h, 0.0)
            out_ref[...] = (jnp.dot(h, lw_ref[...],
                                    preferred_element_type=jnp.float32)
                            + lb_ref[0])
        else:
            out_ref[...] = h

    full = lambda s: pl.BlockSpec(s, lambda i: (0,) * len(s))
    in_specs = [pl.BlockSpec((BN, din), lambda i: (i, 0))]
    operands = [x]
    for arr in sums_list:
        for r in range(R):
            in_specs.append(pl.BlockSpec(
                (NC, BN, 32), lambda i, r=r: (0, r * (NPAD // BN) + i, 0)))
            operands.append(arr)
    for r in range(R):
        in_specs.append(pl.BlockSpec(
            (NC, BN, 16), lambda i, r=r: (0, r * (NPAD // BN) + i, 0)))
        operands.append(cnts)
    in_specs += [full(basis.shape), full(comp.shape), full(root.shape),
                 full(bx.shape), full(bh.shape)]
    operands += [basis, comp, root, bx, bh]
    if final:
        in_specs += [full(lin_w.shape), full(lin_b.shape)]
        operands += [lin_w, lin_b]

    return pl.pallas_call(
        body,
        grid=grid,
        in_specs=in_specs,
        out_specs=pl.BlockSpec((BN, out_w), lambda i: (i, 0)),
        out_shape=jax.ShapeDtypeStruct((N, out_w), jnp.float32),
    )(*operands)


def kernel(x, edge_index, edge_weight, l1x_basis, l1x_comp, l1x_root,
           l1x_bias, l1h_basis, l1h_comp, l1h_root, l1h_bias, l2x_basis,
           l2x_comp, l2x_root, l2x_bias, l2h_basis, l2h_comp, l2h_root,
           l2h_bias, lin_w, lin_b):
    npad_e = E_PAD - E
    src = jnp.concatenate([edge_index[0], jnp.zeros((npad_e,), jnp.int32)])
    dst = jnp.concatenate([edge_index[1],
                           jnp.full((npad_e,), N, jnp.int32)])
    et = jnp.concatenate([edge_weight, jnp.zeros((npad_e,), jnp.int32)])

    b1 = jnp.pad(l1x_basis, ((0, 0), (0, 0), (0, 0), (0, 14)))
    r1 = jnp.pad(l1x_root, ((0, 0), (0, 0), (0, 14)))
    bx1 = jnp.pad(l1x_bias, ((0, 0), (0, 14)))
    bh1 = jnp.pad(l1h_bias, ((0, 0), (0, 14)))
    b2 = jnp.pad(l2x_basis, ((0, 0), (0, 0), (0, 14), (0, 12)))
    r2 = jnp.pad(l2x_root, ((0, 0), (0, 14), (0, 12)))
    bx2 = jnp.pad(l2x_bias, ((0, 0), (0, 12)))
    bh2 = jnp.pad(l2h_bias, ((0, 0), (0, 12)))
    lw = jnp.pad(lin_w, ((0, 12), (0, 0)))

    cnts = _cnt_pass(dst, et)
    s1a = _seg_pass(x[:, :32], x[:, 32:64], src, dst, et)
    s1b = _seg_pass(x[:, 64:96], x[:, 96:], src, dst, et)
    h1p = _gates_kernel(x, [s1a, s1b], cnts, b1, l1x_comp, r1, bx1, bh1,
                        dout=64)

    sums2 = _seg_pass(h1p[:, :32], h1p[:, 32:], src, dst, et)
    out = _gates_kernel(h1p, [sums2], cnts, b2, l2x_comp, r2, bx2, bh2,
                        dout=32, lin_w=lw, lin_b=lin_b)
    return out


# speedup vs baseline: 1.9131x; 1.7121x over previous
import functools

import jax
import jax.numpy as jnp
from jax import lax
from jax.experimental import pallas as pl
from jax.experimental.pallas import tpu as pltpu
from jax.experimental.pallas import tpu_sc as plsc

N = 10000
NPAD = 10240
R = 3
E = 320000
CH = 80
K = 5
GRP = K * CH
NC = 2
NS = 16
ROWS = R * NPAD
RPT = ROWS // NS
BN = 1024


def _seg_pass(tab_lo, tab_hi, ei, ew):
    epw = E // NS
    G = epw // GRP
    mesh = plsc.VectorSubcoreMesh(core_axis_name="c", subcore_axis_name="s")

    scratch = [
        pltpu.VMEM((2, GRP), jnp.int32),
        pltpu.VMEM((2, GRP), jnp.int32),
        pltpu.VMEM((2, GRP), jnp.int32),
        pltpu.VMEM((2, K, CH), jnp.int32),
        pltpu.VMEM((2, K, CH, 32), jnp.float32),
        pltpu.VMEM((CH, 32), jnp.float32),
        pltpu.VMEM_SHARED((ROWS, 32), jnp.float32),
        pltpu.SemaphoreType.DMA,
        pltpu.SemaphoreType.DMA,
        pltpu.SemaphoreType.DMA,
    ]

    @functools.partial(
        pl.kernel,
        out_type=jax.ShapeDtypeStruct((NC, ROWS, 32), jnp.float32),
        mesh=mesh,
        compiler_params=pltpu.CompilerParams(use_tc_tiling_on_sc=False),
        scratch_types=scratch,
    )
    def k(tlo, thi, ei_r, ew_r, out_s, gidx, dvec, evec, sidx, rows, zbuf,
          accum, sem_i, sem_g, sem_s):
        cid = lax.axis_index("c")
        sid = lax.axis_index("s")
        ebase = sid * epw

        def zrow(i, _):
            zbuf[i, pl.ds(0, 16)] = jnp.zeros((16,), jnp.float32)
            zbuf[i, pl.ds(16, 16)] = jnp.zeros((16,), jnp.float32)
            return _
        lax.fori_loop(0, CH, zrow, None)

        def zfire(t, _):
            pltpu.async_copy(zbuf, accum.at[pl.ds(sid * RPT + t * CH, CH)],
                             sem_i)
            return _
        lax.fori_loop(0, RPT // CH, zfire, None)

        def zwait(t, _):
            pltpu.make_async_copy(zbuf, accum.at[pl.ds(sid * RPT, CH)],
                                  sem_i).wait()
            return _
        lax.fori_loop(0, RPT // CH, zwait, None)
        plsc.subcore_barrier()

        def idx_fire(g, pb):
            b = ebase + g * GRP
            pltpu.async_copy(ei_r.at[0, pl.ds(b, GRP)], gidx.at[pb], sem_i)
            pltpu.async_copy(ei_r.at[1, pl.ds(b, GRP)], dvec.at[pb], sem_i)
            pltpu.async_copy(ew_r.at[pl.ds(b, GRP)], evec.at[pb], sem_i)

        def idx_wait(pb):
            pltpu.make_async_copy(ei_r.at[0, pl.ds(0, GRP)], gidx.at[pb],
                                  sem_i).wait()
            pltpu.make_async_copy(ei_r.at[0, pl.ds(0, GRP)], dvec.at[pb],
                                  sem_i).wait()
            pltpu.make_async_copy(ew_r.at[pl.ds(0, GRP)], evec.at[pb],
                                  sem_i).wait()

        def idx_compute(pb):
            for j in range(K):
                for i in range(CH // 16):
                    o = j * CH + i * 16
                    d = dvec[pb, pl.ds(o, 16)]
                    e = evec[pb, pl.ds(o, 16)]
                    sidx[pb, j, pl.ds(i * 16, 16)] = e * NPAD + d

        def gather_fire(pb, j):
            idx = gidx.at[pb, pl.ds(j * CH, CH)]

            @pl.when(cid == 0)
            def _():
                pltpu.async_copy(tlo.at[idx], rows.at[pb, j], sem_g)

            @pl.when(cid == 1)
            def _():
                pltpu.async_copy(thi.at[idx], rows.at[pb, j], sem_g)

        def gather_wait(pb, j):
            idx = gidx.at[pb, pl.ds(j * CH, CH)]
            pltpu.make_async_copy(tlo.at[idx], rows.at[pb, j], sem_g).wait()

        def scat_fire(pb, j):
            pltpu.async_copy(rows.at[pb, j], accum.at[sidx.at[pb, j]],
                             sem_s, add=True)

        def scat_wait(pb, j):
            pltpu.make_async_copy(rows.at[pb, j], accum.at[sidx.at[pb, j]],
                                  sem_s).wait()

        idx_fire(0, 0)
        idx_wait(0)
        idx_compute(0)
        idx_fire(1, 1)
        for j in range(K):
            gather_fire(0, j)
        idx_wait(1)
        idx_compute(1)
        for j in range(K):
            gather_wait(0, j)
            scat_fire(0, j)
        idx_fire(2, 0)
        for j in range(K):
            gather_fire(1, j)

        def pair(t, _):
            for pb in range(2):
                g = 2 + 2 * t + pb
                idx_wait(pb)
                for j in range(K):
                    scat_wait(pb, j)
                idx_compute(pb)
                for j in range(K):
                    gather_wait(1 - pb, j)
                    scat_fire(1 - pb, j)
                gnext = jnp.minimum(g + 1, G - 1)
                idx_fire(gnext, 1 - pb)
                for j in range(K):
                    gather_fire(pb, j)
            return _
        lax.fori_loop(0, (G - 2) // 2, pair, None)

        idx_wait(0)
        for j in range(K):
            scat_wait(0, j)
        for j in range(K):
            gather_wait(1, j)
            scat_fire(1, j)
        for j in range(K):
            scat_wait(1, j)
        plsc.subcore_barrier()

        sl = pl.ds(sid * RPT, RPT)
        pltpu.sync_copy(accum.at[sl], out_s.at[cid, sl])

    return k(tab_lo, tab_hi, ei, ew)


def _cnt_pass(ei, ew):
    epw = E // (NC * NS)
    G = epw // GRP
    mesh = plsc.VectorSubcoreMesh(core_axis_name="c", subcore_axis_name="s")

    @functools.partial(
        pl.kernel,
        out_type=jax.ShapeDtypeStruct((NC, ROWS, 16), jnp.float32),
        mesh=mesh,
        compiler_params=pltpu.CompilerParams(use_tc_tiling_on_sc=False),
        scratch_types=[
            pltpu.VMEM((2, GRP), jnp.int32),
            pltpu.VMEM((2, GRP), jnp.int32),
            pltpu.VMEM((2, K, CH), jnp.int32),
            pltpu.VMEM((CH, 16), jnp.float32),
            pltpu.VMEM((CH, 16), jnp.float32),
            pltpu.VMEM_SHARED((ROWS, 16), jnp.float32),
            pltpu.SemaphoreType.DMA,
            pltpu.SemaphoreType.DMA,
        ],
    )
    def k(ei_r, ew_r, out_c, dvec, evec, sidx, ones, zbuf, accum, sem_i,
          sem_c):
        cid = lax.axis_index("c")
        sid = lax.axis_index("s")
        ebase = (sid * NC + cid) * epw

        def zrow(i, _):
            ones[i, pl.ds(0, 16)] = jnp.ones((16,), jnp.float32)
            zbuf[i, pl.ds(0, 16)] = jnp.zeros((16,), jnp.float32)
            return _
        lax.fori_loop(0, CH, zrow, None)

        def zfire(t, _):
            pltpu.async_copy(zbuf, accum.at[pl.ds(sid * RPT + t * CH, CH)],
                             sem_i)
            return _
        lax.fori_loop(0, RPT // CH, zfire, None)

        def zwait(t, _):
            pltpu.make_async_copy(zbuf, accum.at[pl.ds(sid * RPT, CH)],
                                  sem_i).wait()
            return _
        lax.fori_loop(0, RPT // CH, zwait, None)
        plsc.subcore_barrier()

        def idx_fire(g, pb):
            b = ebase + g * GRP
            pltpu.async_copy(ei_r.at[1, pl.ds(b, GRP)], dvec.at[pb], sem_i)
            pltpu.async_copy(ew_r.at[pl.ds(b, GRP)], evec.at[pb], sem_i)

        def idx_wait(pb):
            pltpu.make_async_copy(ei_r.at[1, pl.ds(0, GRP)], dvec.at[pb],
                                  sem_i).wait()
            pltpu.make_async_copy(ew_r.at[pl.ds(0, GRP)], evec.at[pb],
                                  sem_i).wait()

        def idx_compute(pb):
            for j in range(K):
                for i in range(CH // 16):
                    o = j * CH + i * 16
                    d = dvec[pb, pl.ds(o, 16)]
                    e = evec[pb, pl.ds(o, 16)]
                    sidx[pb, j, pl.ds(i * 16, 16)] = e * NPAD + d

        def scat_fire(pb, j):
            pltpu.async_copy(ones, accum.at[sidx.at[pb, j]], sem_c,
                             add=True)

        def scat_wait(pb, j):
            pltpu.make_async_copy(ones, accum.at[sidx.at[pb, j]],
                                  sem_c).wait()

        def one(g, pb):
            idx_wait(pb)
            for j in range(K):
                scat_wait(pb, j)
            idx_compute(pb)
            for j in range(K):
                scat_fire(pb, j)
            gnext = jnp.minimum(g + 1, G - 1)
            idx_fire(gnext, 1 - pb)

        idx_fire(0, 0)
        idx_wait(0)
        idx_compute(0)
        idx_fire(1, 1)
        for j in range(K):
            scat_fire(0, j)
        idx_wait(1)
        idx_compute(1)
        for j in range(K):
            scat_fire(1, j)
        idx_fire(2, 0)

        g0 = 2
        if (G - 2) % 2 == 1:
            one(2, 0)
            g0 = 3

        def pair(t, _):
            for i in range(2):
                one(g0 + 2 * t + i, (g0 + i) & 1)
            return _
        lax.fori_loop(0, (G - g0) // 2, pair, None)

        pl_ = (G - 1) & 1
        idx_wait(1 - pl_)
        for j in range(K):
            scat_wait(1 - pl_, j)
        for j in range(K):
            scat_wait(pl_, j)
        plsc.subcore_barrier()

        sl = pl.ds(sid * RPT, RPT)
        pltpu.sync_copy(accum.at[sl], out_c.at[cid, sl])

    return k(ei, ew)


def _inv_kernel(cnts):
    def body(c_ref, o_ref):
        c = c_ref[0, :, 0] + c_ref[1, :, 0]
        o_ref[...] = 1.0 / jnp.maximum(c, 1.0)

    return pl.pallas_call(
        body,
        grid=(ROWS // BN,),
        in_specs=[pl.BlockSpec((NC, BN, 16), lambda i: (0, i, 0))],
        out_specs=pl.BlockSpec((BN,), lambda i: (i,)),
        out_shape=jax.ShapeDtypeStruct((ROWS,), jnp.float32),
    )(cnts)


def _gates_kernel(xs, sums_list, inv, basis, comp, root, bx, bh, dout,
                  lin_w=None, lin_b=None):
    grid = (NPAD // BN,)
    final = lin_w is not None
    nx = len(xs)
    nsum = len(sums_list)

    def body(*refs):
        x_refs = refs[:nx]
        i = nx
        sums_refs = refs[i:i + 3 * nsum]
        i += 3 * nsum
        inv_refs = refs[i:i + 3]
        i += 3
        basis_ref, comp_ref, root_ref, bx_ref, bh_ref = refs[i:i + 5]
        i += 5
        if final:
            lw_ref, lb_ref = refs[i:i + 2]
            out_refs = refs[-1:]
        else:
            out_refs = refs[-2:]

        pieces = [xr[...] for xr in x_refs]
        for r in range(R):
            iv = inv_refs[r][...][:, None]
            for a in range(nsum):
                sref = sums_refs[a * 3 + r]
                pieces.append(sref[0] * iv)
                pieces.append(sref[1] * iv)
        mcat = jnp.concatenate(pieces, axis=1)

        wcols = []
        bparts = []
        for k in (0, 2, 3):
            rows = [root_ref[k]]
            for r in range(R):
                rows.append(comp_ref[k, r, 0] * basis_ref[k, 0]
                            + comp_ref[k, r, 1] * basis_ref[k, 1]
                            + comp_ref[k, r, 2] * basis_ref[k, 2])
            wcols.append(jnp.concatenate(rows, axis=0))
            bparts.append(bx_ref[k] + bh_ref[k])
        wall = jnp.concatenate(wcols, axis=1)
        bias = jnp.concatenate(bparts, axis=0)

        z = jnp.dot(mcat, wall, preferred_element_type=jnp.float32) + bias
        gi = jax.nn.sigmoid(z[:, :dout])
        gt = jnp.tanh(z[:, dout:2 * dout])
        go = jax.nn.sigmoid(z[:, 2 * dout:])
        h = go * jnp.tanh(gi * gt)
        if final:
            h = jnp.maximum(h, 0.0)
            out_refs[0][...] = (jnp.dot(h, lw_ref[...],
                                        preferred_element_type=jnp.float32)
                                + lb_ref[0])
        else:
            half = dout // 2
            out_refs[0][...] = h[:, :half]
            out_refs[1][...] = h[:, half:]

    full = lambda s: pl.BlockSpec(s, lambda i: (0,) * len(s))
    in_specs = [pl.BlockSpec((BN, xa.shape[1]), lambda i: (i, 0))
                for xa in xs]
    operands = list(xs)
    for arr in sums_list:
        for r in range(R):
            in_specs.append(pl.BlockSpec(
                (NC, BN, 32), lambda i, r=r: (0, r * (NPAD // BN) + i, 0)))
            operands.append(arr)
    for r in range(R):
        in_specs.append(pl.BlockSpec(
            (BN,), lambda i, r=r: (r * (NPAD // BN) + i,)))
        operands.append(inv)
    in_specs += [full(basis.shape), full(comp.shape), full(root.shape),
                 full(bx.shape), full(bh.shape)]
    operands += [basis, comp, root, bx, bh]
    if final:
        in_specs += [full(lin_w.shape), full(lin_b.shape)]
        operands += [lin_w, lin_b]
        out_specs = pl.BlockSpec((BN, 1), lambda i: (i, 0))
        out_shape = jax.ShapeDtypeStruct((N, 1), jnp.float32)
    else:
        half = dout // 2
        out_specs = [pl.BlockSpec((BN, half), lambda i: (i, 0))] * 2
        out_shape = [jax.ShapeDtypeStruct((N, half), jnp.float32)] * 2

    return pl.pallas_call(
        body,
        grid=grid,
        in_specs=in_specs,
        out_specs=out_specs,
        out_shape=out_shape,
    )(*operands)


def kernel(x, edge_index, edge_weight, l1x_basis, l1x_comp, l1x_root,
           l1x_bias, l1h_basis, l1h_comp, l1h_root, l1h_bias, l2x_basis,
           l2x_comp, l2x_root, l2x_bias, l2h_basis, l2h_comp, l2h_root,
           l2h_bias, lin_w, lin_b):
    b1 = jnp.pad(l1x_basis, ((0, 0), (0, 0), (0, 0), (0, 14)))
    r1 = jnp.pad(l1x_root, ((0, 0), (0, 0), (0, 14)))
    bx1 = jnp.pad(l1x_bias, ((0, 0), (0, 14)))
    bh1 = jnp.pad(l1h_bias, ((0, 0), (0, 14)))
    b2 = jnp.pad(l2x_basis, ((0, 0), (0, 0), (0, 14), (0, 12)))
    r2 = jnp.pad(l2x_root, ((0, 0), (0, 14), (0, 12)))
    bx2 = jnp.pad(l2x_bias, ((0, 0), (0, 12)))
    bh2 = jnp.pad(l2h_bias, ((0, 0), (0, 12)))
    lw = jnp.pad(lin_w, ((0, 12), (0, 0)))

    cnts = _cnt_pass(edge_index, edge_weight)
    inv = _inv_kernel(cnts)
    czero = (cnts[0, 0, 1] * 0.0).astype(x.dtype)
    s1a = _seg_pass(x[:, :32] + czero, x[:, 32:64], edge_index, edge_weight)
    s1b = _seg_pass(x[:, 64:96], x[:, 96:], edge_index, edge_weight)
    h1lo, h1hi = _gates_kernel([x], [s1a, s1b], inv, b1, l1x_comp, r1,
                               bx1, bh1, dout=64)

    sums2 = _seg_pass(h1lo, h1hi, edge_index, edge_weight)
    out = _gates_kernel([h1lo, h1hi], [sums2], inv, b2, l2x_comp, r2,
                        bx2, bh2, dout=32, lin_w=lw, lin_b=lin_b)
    return out
